# baseline (speedup 1.0000x reference)
import jax
import jax.numpy as jnp
from jax.experimental import pallas as pl
from jax.experimental.pallas import tpu as pltpu

N = 4096
NFEAT = 128
NHID = 64
NCLASS = 16
NLAYERS = 14
EPS = 1e-5
BLK = 256
NBLK = N // BLK
INV_N = 1.0 / N


def _gcn_kernel(x_ref, adj_ref, W1_ref, Wh_ref, Wend_ref, bend_ref,
                g_ref, be_ref, o_ref,
                adj_bf, out_scr, y_scr, s_scr, stat_scr):
    r = pl.program_id(0)

    @pl.when(r == 0)
    def _():
        s_scr[...] = jnp.dot(x_ref[...].astype(jnp.bfloat16),
                             W1_ref[...].astype(jnp.bfloat16),
                             preferred_element_type=jnp.float32).astype(jnp.bfloat16)
        stat_scr[...] = jnp.zeros_like(stat_scr)
        y_scr[...] = jnp.zeros_like(y_scr)

    aw = adj_ref[...]
    ob = jnp.dot(aw, s_scr[...].astype(jnp.float32),
                 preferred_element_type=jnp.float32)
    out_scr[pl.ds(r * BLK, BLK), :] = ob
    stat_scr[0:1, :] += jnp.sum(ob, axis=0, keepdims=True)
    stat_scr[1:2, :] += jnp.sum(ob * ob, axis=0, keepdims=True)
    adj_bf[pl.ds(r * BLK, BLK), :] = aw.astype(jnp.bfloat16)

    def bn_coeffs(ssum, ssq, g, be):
        mu = ssum * INV_N
        var = ssq * INV_N - mu * mu
        a = g * jax.lax.rsqrt(var + EPS)
        return a, be - mu * a

    @pl.when(r == NBLK - 1)
    def _():
        def norm_support(a, d, W_bf):
            def nb(rr, _):
                sl = pl.ds(rr * BLK, BLK)
                yb = jnp.maximum(out_scr[sl, :] * a + d, 0.0) + y_scr[sl, :]
                y_scr[sl, :] = yb
                s_scr[sl, :] = jnp.dot(
                    yb.astype(jnp.bfloat16), W_bf,
                    preferred_element_type=jnp.float32).astype(jnp.bfloat16)
                return 0
            jax.lax.fori_loop(0, NBLK, nb, 0, unroll=8)

        def adj_mm_stats():
            sv = s_scr[...]
            def blk(rr, carry):
                ssum, ssq = carry
                ab = adj_bf[pl.ds(rr * BLK, BLK), :]
                ob = jnp.dot(ab, sv, preferred_element_type=jnp.float32)
                out_scr[pl.ds(rr * BLK, BLK), :] = ob
                return (ssum + jnp.sum(ob, axis=0, keepdims=True),
                        ssq + jnp.sum(ob * ob, axis=0, keepdims=True))
            z = jnp.zeros((1, NHID), jnp.float32)
            return jax.lax.fori_loop(0, NBLK, blk, (z, z), unroll=8)

        a0, d0 = bn_coeffs(stat_scr[0:1, :], stat_scr[1:2, :],
                           g_ref[0:1, :], be_ref[0:1, :])

        def layer(i, carry):
            a, d = carry
            norm_support(a, d, Wh_ref[i].astype(jnp.bfloat16))
            ssum, ssq = adj_mm_stats()
            return bn_coeffs(ssum, ssq, g_ref[pl.ds(i + 1, 1), :],
                             be_ref[pl.ds(i + 1, 1), :])

        a, d = jax.lax.fori_loop(0, NLAYERS, layer, (a0, d0), unroll=False)

        norm_support(a, d, Wend_ref[...].astype(jnp.bfloat16))
        sv = s_scr[:, :NCLASS]

        def blk_end(rr, _):
            ab = adj_bf[pl.ds(rr * BLK, BLK), :]
            o_ref[pl.ds(rr * BLK, BLK), :] = jax.nn.sigmoid(
                jnp.dot(ab, sv, preferred_element_type=jnp.float32)
                + bend_ref[0:1, :])
            return 0

        jax.lax.fori_loop(0, NBLK, blk_end, 0, unroll=4)


def kernel(x, adj, W1, b1, Wh, bh, Wend, bend, gamma, beta):
    x2 = x[0]
    full = lambda shape: pl.BlockSpec(shape, lambda r: (0,) * len(shape),
                                      memory_space=pltpu.MemorySpace.VMEM)
    Wend_pad = jnp.zeros((NHID, NHID), jnp.float32).at[:, :NCLASS].set(Wend)
    out = pl.pallas_call(
        _gcn_kernel,
        grid=(NBLK,),
        out_shape=jax.ShapeDtypeStruct((N, NCLASS), jnp.float32),
        in_specs=[
            full((N, NFEAT)),
            pl.BlockSpec((BLK, N), lambda r: (r, 0),
                         memory_space=pltpu.MemorySpace.VMEM),
            full((NFEAT, NHID)),
            full((NLAYERS, NHID, NHID)),
            full((NHID, NHID)),
            full((1, NCLASS)),
            full((15, NHID)),
            full((15, NHID)),
        ],
        out_specs=full((N, NCLASS)),
        scratch_shapes=[pltpu.VMEM((N, N), jnp.bfloat16),
                        pltpu.VMEM((N, NHID), jnp.float32),
                        pltpu.VMEM((N, NHID), jnp.float32),
                        pltpu.VMEM((N, NHID), jnp.bfloat16),
                        pltpu.VMEM((2, NHID), jnp.float32)],
        compiler_params=pltpu.CompilerParams(
            vmem_limit_bytes=100 * 1024 * 1024,
        ),
    )(x2, adj, W1, Wh, Wend_pad, bend.reshape(1, NCLASS), gamma, beta)
    return out[None]


# speedup vs baseline: 1.0328x; 1.0328x over previous
import jax
import jax.numpy as jnp
from jax.experimental import pallas as pl
from jax.experimental.pallas import tpu as pltpu

N = 4096
NFEAT = 128
NHID = 64
NCLASS = 16
NLAYERS = 14
EPS = 1e-5
BLK = 256
NBLK = N // BLK
INV_N = 1.0 / N


def _gcn_kernel(x_ref, adj_ref, W1_ref, Wh_ref, Wend_ref, bend_ref,
                g_ref, be_ref, o_ref,
                adj_bf, out_scr, y_scr, s_scr, stat_scr):
    r = pl.program_id(0)

    @pl.when(r == 0)
    def _():
        s_scr[...] = jnp.dot(x_ref[...].astype(jnp.bfloat16),
                             W1_ref[...].astype(jnp.bfloat16),
                             preferred_element_type=jnp.float32)
        stat_scr[...] = jnp.zeros_like(stat_scr)

    aw = adj_ref[...]
    ob = jnp.dot(aw, s_scr[...], preferred_element_type=jnp.float32)
    out_scr[pl.ds(r * BLK, BLK), :] = ob
    stat_scr[0:1, :] += jnp.sum(ob, axis=0, keepdims=True)
    stat_scr[1:2, :] += jnp.sum(ob * ob, axis=0, keepdims=True)
    adj_bf[pl.ds(r * BLK, BLK), :] = aw.astype(jnp.bfloat16)

    def bn_coeffs(ssum, ssq, g, be):
        mu = ssum * INV_N
        var = ssq * INV_N - mu * mu
        a = g * jax.lax.rsqrt(var + EPS)
        return a, be - mu * a

    @pl.when(r == NBLK - 1)
    def _():
        a, d = bn_coeffs(stat_scr[0:1, :], stat_scr[1:2, :],
                         g_ref[0:1, :], be_ref[0:1, :])
        y_scr[...] = jnp.maximum(out_scr[...] * a + d, 0.0)

        def adj_mm_stats(s_bf):
            def blk(rr, carry):
                ssum, ssq = carry
                ab = adj_bf[pl.ds(rr * BLK, BLK), :]
                ob = jnp.dot(ab, s_bf, preferred_element_type=jnp.float32)
                out_scr[pl.ds(rr * BLK, BLK), :] = ob
                return (ssum + jnp.sum(ob, axis=0, keepdims=True),
                        ssq + jnp.sum(ob * ob, axis=0, keepdims=True))
            z = jnp.zeros((1, NHID), jnp.float32)
            return jax.lax.fori_loop(0, NBLK, blk, (z, z), unroll=16)

        def layer(i, _):
            s = jnp.dot(y_scr[...].astype(jnp.bfloat16),
                        Wh_ref[i].astype(jnp.bfloat16),
                        preferred_element_type=jnp.float32)
            ssum, ssq = adj_mm_stats(s.astype(jnp.bfloat16))
            a, d = bn_coeffs(ssum, ssq, g_ref[pl.ds(i + 1, 1), :],
                             be_ref[pl.ds(i + 1, 1), :])
            y_scr[...] = jnp.maximum(out_scr[...] * a + d, 0.0) + y_scr[...]
            return 0

        jax.lax.fori_loop(0, NLAYERS, layer, 0, unroll=False)

        s = jnp.dot(y_scr[...].astype(jnp.bfloat16),
                    Wend_ref[...].astype(jnp.bfloat16),
                    preferred_element_type=jnp.float32).astype(jnp.bfloat16)

        def blk_end(rr, _):
            ab = adj_bf[pl.ds(rr * BLK, BLK), :]
            o_ref[pl.ds(rr * BLK, BLK), :] = jax.nn.sigmoid(
                jnp.dot(ab, s, preferred_element_type=jnp.float32)
                + bend_ref[0:1, :])
            return 0

        jax.lax.fori_loop(0, NBLK, blk_end, 0, unroll=8)


def kernel(x, adj, W1, b1, Wh, bh, Wend, bend, gamma, beta):
    x2 = x[0]
    full = lambda shape: pl.BlockSpec(shape, lambda r: (0,) * len(shape),
                                      memory_space=pltpu.MemorySpace.VMEM)
    out = pl.pallas_call(
        _gcn_kernel,
        grid=(NBLK,),
        out_shape=jax.ShapeDtypeStruct((N, NCLASS), jnp.float32),
        in_specs=[
            full((N, NFEAT)),
            pl.BlockSpec((BLK, N), lambda r: (r, 0),
                         memory_space=pltpu.MemorySpace.VMEM),
            full((NFEAT, NHID)),
            full((NLAYERS, NHID, NHID)),
            full((NHID, NCLASS)),
            full((1, NCLASS)),
            full((15, NHID)),
            full((15, NHID)),
        ],
        out_specs=full((N, NCLASS)),
        scratch_shapes=[pltpu.VMEM((N, N), jnp.bfloat16),
                        pltpu.VMEM((N, NHID), jnp.float32),
                        pltpu.VMEM((N, NHID), jnp.float32),
                        pltpu.VMEM((N, NHID), jnp.float32),
                        pltpu.VMEM((2, NHID), jnp.float32)],
        compiler_params=pltpu.CompilerParams(
            vmem_limit_bytes=100 * 1024 * 1024,
        ),
    )(x2, adj, W1, Wh, Wend, bend.reshape(1, NCLASS), gamma, beta)
    return out[None]


# speedup vs baseline: 1.0698x; 1.0359x over previous
import jax
import jax.numpy as jnp
from jax.experimental import pallas as pl
from jax.experimental.pallas import tpu as pltpu

N = 4096
NFEAT = 128
NHID = 64
NCLASS = 16
NLAYERS = 14
EPS = 1e-5
BLK = 512
NBLK = N // BLK
INV_N = 1.0 / N


def _gcn_kernel(x_ref, adj_ref, W1_ref, Wh_ref, Wend_ref, bend_ref,
                g_ref, be_ref, o_ref,
                adj_bf, out_scr, y_scr, s_scr, stat_scr):
    r = pl.program_id(0)

    @pl.when(r == 0)
    def _():
        s_scr[...] = jnp.dot(x_ref[...].astype(jnp.bfloat16),
                             W1_ref[...].astype(jnp.bfloat16),
                             preferred_element_type=jnp.float32)
        stat_scr[...] = jnp.zeros_like(stat_scr)

    aw = adj_ref[...]
    ob = jnp.dot(aw, s_scr[...], preferred_element_type=jnp.float32)
    out_scr[pl.ds(r * BLK, BLK), :] = ob
    stat_scr[0:1, :] += jnp.sum(ob, axis=0, keepdims=True)
    stat_scr[1:2, :] += jnp.sum(ob * ob, axis=0, keepdims=True)
    adj_bf[pl.ds(r * BLK, BLK), :] = aw.astype(jnp.bfloat16)

    def bn_coeffs(ssum, ssq, g, be):
        mu = ssum * INV_N
        var = ssq * INV_N - mu * mu
        a = g * jax.lax.rsqrt(var + EPS)
        return a, be - mu * a

    @pl.when(r == NBLK - 1)
    def _():
        a, d = bn_coeffs(stat_scr[0:1, :], stat_scr[1:2, :],
                         g_ref[0:1, :], be_ref[0:1, :])
        y_scr[...] = jnp.maximum(out_scr[...] * a + d, 0.0)

        def adj_mm_stats(s_bf):
            def blk(rr, carry):
                ssum, ssq = carry
                ab = adj_bf[pl.ds(rr * BLK, BLK), :]
                ob = jnp.dot(ab, s_bf, preferred_element_type=jnp.float32)
                out_scr[pl.ds(rr * BLK, BLK), :] = ob
                return (ssum + jnp.sum(ob, axis=0, keepdims=True),
                        ssq + jnp.sum(ob * ob, axis=0, keepdims=True))
            z = jnp.zeros((1, NHID), jnp.float32)
            return jax.lax.fori_loop(0, NBLK, blk, (z, z), unroll=8)

        def layer(i, _):
            s = jnp.dot(y_scr[...].astype(jnp.bfloat16),
                        Wh_ref[i].astype(jnp.bfloat16),
                        preferred_element_type=jnp.float32)
            ssum, ssq = adj_mm_stats(s.astype(jnp.bfloat16))
            a, d = bn_coeffs(ssum, ssq, g_ref[pl.ds(i + 1, 1), :],
                             be_ref[pl.ds(i + 1, 1), :])
            y_scr[...] = jnp.maximum(out_scr[...] * a + d, 0.0) + y_scr[...]
            return 0

        jax.lax.fori_loop(0, NLAYERS, layer, 0, unroll=False)

        s = jnp.dot(y_scr[...].astype(jnp.bfloat16),
                    Wend_ref[...].astype(jnp.bfloat16),
                    preferred_element_type=jnp.float32).astype(jnp.bfloat16)

        def blk_end(rr, _):
            ab = adj_bf[pl.ds(rr * BLK, BLK), :]
            o_ref[pl.ds(rr * BLK, BLK), :] = jax.nn.sigmoid(
                jnp.dot(ab, s, preferred_element_type=jnp.float32)
                + bend_ref[0:1, :])
            return 0

        jax.lax.fori_loop(0, NBLK, blk_end, 0, unroll=8)


def kernel(x, adj, W1, b1, Wh, bh, Wend, bend, gamma, beta):
    x2 = x[0]
    full = lambda shape: pl.BlockSpec(shape, lambda r: (0,) * len(shape),
                                      memory_space=pltpu.MemorySpace.VMEM)
    out = pl.pallas_call(
        _gcn_kernel,
        grid=(NBLK,),
        out_shape=jax.ShapeDtypeStruct((N, NCLASS), jnp.float32),
        in_specs=[
            full((N, NFEAT)),
            pl.BlockSpec((BLK, N), lambda r: (r, 0),
                         memory_space=pltpu.MemorySpace.VMEM),
            full((NFEAT, NHID)),
            full((NLAYERS, NHID, NHID)),
            full((NHID, NCLASS)),
            full((1, NCLASS)),
            full((15, NHID)),
            full((15, NHID)),
        ],
        out_specs=full((N, NCLASS)),
        scratch_shapes=[pltpu.VMEM((N, N), jnp.bfloat16),
                        pltpu.VMEM((N, NHID), jnp.float32),
                        pltpu.VMEM((N, NHID), jnp.float32),
                        pltpu.VMEM((N, NHID), jnp.float32),
                        pltpu.VMEM((2, NHID), jnp.float32)],
        compiler_params=pltpu.CompilerParams(
            vmem_limit_bytes=100 * 1024 * 1024,
        ),
    )(x2, adj, W1, Wh, Wend, bend.reshape(1, NCLASS), gamma, beta)
    return out[None]


# speedup vs baseline: 1.0830x; 1.0123x over previous
import jax
import jax.numpy as jnp
from jax.experimental import pallas as pl
from jax.experimental.pallas import tpu as pltpu

N = 4096
NFEAT = 128
NHID = 64
NCLASS = 16
NLAYERS = 14
EPS = 1e-5
BLK = 512
NBLK = N // BLK
INV_N = 1.0 / N


def _gcn_kernel(x_ref, adj_ref, W1_ref, Wh_ref, Wend_ref, bend_ref,
                g_ref, be_ref, o_ref,
                adj_bf, out_scr, y_scr, s_scr, stat_scr):
    r = pl.program_id(0)

    @pl.when(r == 0)
    def _():
        s_scr[...] = jnp.dot(x_ref[...].astype(jnp.bfloat16),
                             W1_ref[...].astype(jnp.bfloat16),
                             preferred_element_type=jnp.float32)
        stat_scr[...] = jnp.zeros_like(stat_scr)

    aw = adj_ref[...]
    ob = jnp.dot(aw, s_scr[...], preferred_element_type=jnp.float32)
    out_scr[pl.ds(r * BLK, BLK), :] = ob
    stat_scr[0:1, :] += jnp.sum(ob, axis=0, keepdims=True)
    stat_scr[1:2, :] += jnp.sum(ob * ob, axis=0, keepdims=True)
    adj_bf[pl.ds(r * BLK, BLK), :] = aw.astype(jnp.bfloat16)

    def bn_coeffs(ssum, ssq, g, be):
        mu = ssum * INV_N
        var = ssq * INV_N - mu * mu
        a = g * jax.lax.rsqrt(var + EPS)
        return a, be - mu * a

    @pl.when(r == NBLK - 1)
    def _():
        a, d = bn_coeffs(stat_scr[0:1, :], stat_scr[1:2, :],
                         g_ref[0:1, :], be_ref[0:1, :])
        y_scr[...] = jnp.maximum(out_scr[...] * a + d, 0.0)

        def adj_mm_stats(s_bf):
            def blk(rr, carry):
                ssum, ssq = carry
                ab = adj_bf[pl.ds(rr * BLK, BLK), :]
                ob = jnp.dot(ab, s_bf, preferred_element_type=jnp.float32)
                out_scr[pl.ds(rr * BLK, BLK), :] = ob
                return (ssum + jnp.sum(ob, axis=0, keepdims=True),
                        ssq + jnp.sum(ob * ob, axis=0, keepdims=True))
            z = jnp.zeros((1, NHID), jnp.float32)
            return jax.lax.fori_loop(0, NBLK, blk, (z, z), unroll=8)

        def layer(i, _):
            s = jnp.dot(y_scr[...].astype(jnp.bfloat16),
                        Wh_ref[i].astype(jnp.bfloat16),
                        preferred_element_type=jnp.float32)
            ssum, ssq = adj_mm_stats(s.astype(jnp.bfloat16))
            a, d = bn_coeffs(ssum, ssq, g_ref[pl.ds(i + 1, 1), :],
                             be_ref[pl.ds(i + 1, 1), :])
            y_scr[...] = jnp.maximum(out_scr[...] * a + d, 0.0) + y_scr[...]
            return 0

        jax.lax.fori_loop(0, NLAYERS, layer, 0, unroll=2)

        s = jnp.dot(y_scr[...].astype(jnp.bfloat16),
                    Wend_ref[...].astype(jnp.bfloat16),
                    preferred_element_type=jnp.float32).astype(jnp.bfloat16)

        def blk_end(rr, _):
            ab = adj_bf[pl.ds(rr * BLK, BLK), :]
            o_ref[pl.ds(rr * BLK, BLK), :] = jax.nn.sigmoid(
                jnp.dot(ab, s, preferred_element_type=jnp.float32)
                + bend_ref[0:1, :])
            return 0

        jax.lax.fori_loop(0, NBLK, blk_end, 0, unroll=8)


def kernel(x, adj, W1, b1, Wh, bh, Wend, bend, gamma, beta):
    x2 = x[0]
    full = lambda shape: pl.BlockSpec(shape, lambda r: (0,) * len(shape),
                                      memory_space=pltpu.MemorySpace.VMEM)
    out = pl.pallas_call(
        _gcn_kernel,
        grid=(NBLK,),
        out_shape=jax.ShapeDtypeStruct((N, NCLASS), jnp.float32),
        in_specs=[
            full((N, NFEAT)),
            pl.BlockSpec((BLK, N), lambda r: (r, 0),
                         memory_space=pltpu.MemorySpace.VMEM),
            full((NFEAT, NHID)),
            full((NLAYERS, NHID, NHID)),
            full((NHID, NCLASS)),
            full((1, NCLASS)),
            full((15, NHID)),
            full((15, NHID)),
        ],
        out_specs=full((N, NCLASS)),
        scratch_shapes=[pltpu.VMEM((N, N), jnp.bfloat16),
                        pltpu.VMEM((N, NHID), jnp.float32),
                        pltpu.VMEM((N, NHID), jnp.float32),
                        pltpu.VMEM((N, NHID), jnp.float32),
                        pltpu.VMEM((2, NHID), jnp.float32)],
        compiler_params=pltpu.CompilerParams(
            vmem_limit_bytes=100 * 1024 * 1024,
        ),
    )(x2, adj, W1, Wh, Wend, bend.reshape(1, NCLASS), gamma, beta)
    return out[None]
